# baseline (speedup 1.0000x reference)
import functools
import math

import jax
import jax.numpy as jnp
from jax import lax
from jax.experimental import pallas as pl
from jax.experimental.pallas import tpu as pltpu
from jax.experimental.pallas import tpu_sc as plsc

NC, NS, LANES = 2, 16, 16
NW = NC * NS
CHUNK = 128


def _sinusoidal_pe(seq_len, d_model):
    position = jnp.arange(seq_len, dtype=jnp.float32)[:, None]
    div_term = jnp.exp(
        jnp.arange(0, d_model, 2, dtype=jnp.float32)
        * -(math.log(10000.0) / d_model))
    angles = position * div_term[None, :]
    pe = jnp.zeros((seq_len, d_model), dtype=jnp.float32)
    pe = pe.at[:, 0::2].set(jnp.sin(angles))
    pe = pe.at[:, 1::2].set(jnp.cos(angles))
    return pe


def _make_sc_kernel(N, D, rows_per_w, SEQ_LEN):
    n_chunks = rows_per_w // CHUNK
    assert n_chunks % 2 == 0
    mesh = plsc.VectorSubcoreMesh(core_axis_name="c", subcore_axis_name="s")

    @functools.partial(
        pl.kernel,
        mesh=mesh,
        out_type=jax.ShapeDtypeStruct((N, D), jnp.float32),
        scratch_types=[
            pltpu.VMEM((rows_per_w,), jnp.int32),
            pltpu.VMEM((rows_per_w,), jnp.int32),
            pltpu.VMEM((CHUNK, D), jnp.float32),
            pltpu.VMEM((CHUNK, D), jnp.float32),
            pltpu.VMEM((CHUNK, D), jnp.float32),
            pltpu.VMEM((CHUNK, D), jnp.float32),
            pltpu.VMEM((CHUNK, D), jnp.float32),
            pltpu.VMEM((CHUNK, D), jnp.float32),
            pltpu.SemaphoreType.DMA,
            pltpu.SemaphoreType.DMA,
            pltpu.SemaphoreType.DMA,
            pltpu.SemaphoreType.DMA,
            pltpu.SemaphoreType.DMA,
            pltpu.SemaphoreType.DMA,
            pltpu.VMEM_SHARED((600, 128), jnp.float32),
        ],
    )
    def sc_kernel(tok_hbm, combo_hbm, seq_hbm, seg_hbm, out_hbm,
                  seq_all, cidx_all, tok0, tok1, comb0, comb1,
                  out0, out1, st0, st1, sc0, sc1, sw0, sw1, combo_sh):
        tok_v = (tok0, tok1)
        comb_v = (comb0, comb1)
        out_v = (out0, out1)
        sem_t = (st0, st1)
        sem_c = (sc0, sc1)
        sem_w = (sw0, sw1)

        wid = lax.axis_index("s") * NC + lax.axis_index("c")
        w_base = wid * rows_per_w

        @pl.when(lax.axis_index("s") == 0)
        def _():
            pltpu.sync_copy(combo_hbm, combo_sh)

        plsc.subcore_barrier()

        pltpu.sync_copy(seq_hbm.at[pl.ds(w_base, rows_per_w)], seq_all)
        pltpu.sync_copy(seg_hbm.at[pl.ds(w_base, rows_per_w)], cidx_all)

        iota16 = jax.lax.iota(jnp.int32, LANES)
        H = CHUNK // 2

        def fetch(b, ci):
            for q in range(CHUNK // LANES):
                qsl = pl.ds(ci * CHUNK + q * LANES, LANES)
                pos = ((w_base + ci * CHUNK + q * LANES) + iota16) % SEQ_LEN
                cidx_all[qsl] = pos * 3 + cidx_all[qsl]
            sl = pl.ds(ci * CHUNK, CHUNK)
            lo = pl.ds(ci * CHUNK, H)
            hi = pl.ds(ci * CHUNK + H, H)
            pltpu.async_copy(tok_hbm.at[seq_all.at[lo]],
                             tok_v[b].at[pl.ds(0, H)], sem_t[b])
            pltpu.async_copy(tok_hbm.at[seq_all.at[hi]],
                             tok_v[b].at[pl.ds(H, H)], sem_t[b])
            pltpu.async_copy(combo_sh.at[cidx_all.at[sl]], comb_v[b],
                             sem_c[b])

        def wait_gathers(b, ci):
            sl = pl.ds(ci * CHUNK, CHUNK)
            lo = pl.ds(ci * CHUNK, H)
            hi = pl.ds(ci * CHUNK + H, H)
            pltpu.make_async_copy(tok_hbm.at[seq_all.at[lo]],
                                  tok_v[b].at[pl.ds(0, H)], sem_t[b]).wait()
            pltpu.make_async_copy(tok_hbm.at[seq_all.at[hi]],
                                  tok_v[b].at[pl.ds(H, H)], sem_t[b]).wait()
            pltpu.make_async_copy(combo_sh.at[cidx_all.at[sl]], comb_v[b],
                                  sem_c[b]).wait()

        def wait_wb(b, base):
            pltpu.make_async_copy(out_v[b], out_hbm.at[pl.ds(base, CHUNK)],
                                  sem_w[b]).wait()

        def add_rows(b, lo, hi):
            @plsc.parallel_loop(lo, hi)
            def row_body(r):
                for j in range(D // LANES):
                    sl = pl.ds(j * LANES, LANES)
                    out_v[b][r, sl] = tok_v[b][r, sl] + comb_v[b][r, sl]

        fetch(0, 0)
        fetch(1, 1)

        def group_body(g, _):
            for b in range(2):
                c = 2 * g + b
                base = w_base + c * CHUNK
                wait_gathers(b, c)

                @pl.when(c >= 2)
                def _():
                    wait_wb(b, base - 2 * CHUNK)

                Q = CHUNK // 4
                for h in range(4):
                    add_rows(b, h * Q, (h + 1) * Q)
                    pltpu.async_copy(
                        out_v[b].at[pl.ds(h * Q, Q)],
                        out_hbm.at[pl.ds(base + h * Q, Q)], sem_w[b])

                @pl.when(c + 2 < n_chunks)
                def _():
                    fetch(b, c + 2)
            return 0

        lax.fori_loop(0, n_chunks // 2, group_body, 0, unroll=False)

        wait_wb(0, w_base + (n_chunks - 2) * CHUNK)
        wait_wb(1, w_base + (n_chunks - 1) * CHUNK)

    return sc_kernel


def kernel(sequence, segment_label, token_table, segment_table):
    B, L = sequence.shape
    V, D = token_table.shape
    N = B * L
    rows_per_w = N // NW

    pe = _sinusoidal_pe(L, D)
    combo = (pe[:, None, :] + segment_table[None, :, :]).reshape(L * 3, D)
    seq_flat = sequence.reshape(-1).astype(jnp.int32)
    seg_flat = segment_label.reshape(-1).astype(jnp.int32)

    out = _make_sc_kernel(N, D, rows_per_w, L)(
        token_table, combo, seq_flat, seg_flat)
    return out.reshape(B, L, D)


# speedup vs baseline: 1.0139x; 1.0139x over previous
import functools
import math

import jax
import jax.numpy as jnp
from jax import lax
from jax.experimental import pallas as pl
from jax.experimental.pallas import tpu as pltpu
from jax.experimental.pallas import tpu_sc as plsc

NC, NS, LANES = 2, 16, 16
NW = NC * NS
CHUNK = 128


def _sinusoidal_pe(seq_len, d_model):
    position = jnp.arange(seq_len, dtype=jnp.float32)[:, None]
    div_term = jnp.exp(
        jnp.arange(0, d_model, 2, dtype=jnp.float32)
        * -(math.log(10000.0) / d_model))
    angles = position * div_term[None, :]
    pe = jnp.zeros((seq_len, d_model), dtype=jnp.float32)
    pe = pe.at[:, 0::2].set(jnp.sin(angles))
    pe = pe.at[:, 1::2].set(jnp.cos(angles))
    return pe


def _make_sc_kernel(N, D, rows_per_w, SEQ_LEN):
    n_chunks = rows_per_w // CHUNK
    assert n_chunks % 2 == 0
    mesh = plsc.VectorSubcoreMesh(core_axis_name="c", subcore_axis_name="s")

    @functools.partial(
        pl.kernel,
        mesh=mesh,
        out_type=jax.ShapeDtypeStruct((N, D), jnp.float32),
        scratch_types=[
            pltpu.VMEM((rows_per_w,), jnp.int32),
            pltpu.VMEM((rows_per_w,), jnp.int32),
            pltpu.VMEM((CHUNK, D), jnp.float32),
            pltpu.VMEM((CHUNK, D), jnp.float32),
            pltpu.VMEM((CHUNK, D), jnp.float32),
            pltpu.VMEM((CHUNK, D), jnp.float32),
            pltpu.VMEM((CHUNK, D), jnp.float32),
            pltpu.VMEM((CHUNK, D), jnp.float32),
            pltpu.SemaphoreType.DMA,
            pltpu.SemaphoreType.DMA,
            pltpu.SemaphoreType.DMA,
            pltpu.SemaphoreType.DMA,
            pltpu.SemaphoreType.DMA,
            pltpu.SemaphoreType.DMA,
            pltpu.VMEM_SHARED((600, 128), jnp.float32),
        ],
    )
    def sc_kernel(tok_hbm, combo_hbm, seq_hbm, seg_hbm, out_hbm,
                  seq_all, cidx_all, tok0, tok1, comb0, comb1,
                  out0, out1, st0, st1, sc0, sc1, sw0, sw1, combo_sh):
        tok_v = (tok0, tok1)
        comb_v = (comb0, comb1)
        out_v = (out0, out1)
        sem_t = (st0, st1)
        sem_c = (sc0, sc1)
        sem_w = (sw0, sw1)

        wid = lax.axis_index("s") * NC + lax.axis_index("c")
        w_base = wid * rows_per_w

        pltpu.async_copy(seq_hbm.at[pl.ds(w_base, rows_per_w)], seq_all,
                         sem_t[0])
        pltpu.async_copy(seg_hbm.at[pl.ds(w_base, rows_per_w)], cidx_all,
                         sem_t[1])

        @pl.when(lax.axis_index("s") == 0)
        def _():
            pltpu.sync_copy(combo_hbm, combo_sh)

        pltpu.make_async_copy(seq_hbm.at[pl.ds(w_base, rows_per_w)], seq_all,
                              sem_t[0]).wait()
        pltpu.make_async_copy(seg_hbm.at[pl.ds(w_base, rows_per_w)], cidx_all,
                              sem_t[1]).wait()
        plsc.subcore_barrier()

        iota16 = jax.lax.iota(jnp.int32, LANES)
        H = CHUNK // 2

        def fetch(b, ci):
            for q in range(CHUNK // LANES):
                qsl = pl.ds(ci * CHUNK + q * LANES, LANES)
                pos = ((w_base + ci * CHUNK + q * LANES) + iota16) % SEQ_LEN
                cidx_all[qsl] = pos * 3 + cidx_all[qsl]
            sl = pl.ds(ci * CHUNK, CHUNK)
            lo = pl.ds(ci * CHUNK, H)
            hi = pl.ds(ci * CHUNK + H, H)
            pltpu.async_copy(tok_hbm.at[seq_all.at[lo]],
                             tok_v[b].at[pl.ds(0, H)], sem_t[b])
            pltpu.async_copy(tok_hbm.at[seq_all.at[hi]],
                             tok_v[b].at[pl.ds(H, H)], sem_t[b])
            pltpu.async_copy(combo_sh.at[cidx_all.at[sl]], comb_v[b],
                             sem_c[b])

        def wait_gathers(b, ci):
            sl = pl.ds(ci * CHUNK, CHUNK)
            lo = pl.ds(ci * CHUNK, H)
            hi = pl.ds(ci * CHUNK + H, H)
            pltpu.make_async_copy(tok_hbm.at[seq_all.at[lo]],
                                  tok_v[b].at[pl.ds(0, H)], sem_t[b]).wait()
            pltpu.make_async_copy(tok_hbm.at[seq_all.at[hi]],
                                  tok_v[b].at[pl.ds(H, H)], sem_t[b]).wait()
            pltpu.make_async_copy(combo_sh.at[cidx_all.at[sl]], comb_v[b],
                                  sem_c[b]).wait()

        def wait_wb(b, base):
            pltpu.make_async_copy(out_v[b], out_hbm.at[pl.ds(base, CHUNK)],
                                  sem_w[b]).wait()

        def add_rows(b, lo, hi):
            @plsc.parallel_loop(lo, hi)
            def row_body(r):
                for j in range(D // LANES):
                    sl = pl.ds(j * LANES, LANES)
                    out_v[b][r, sl] = tok_v[b][r, sl] + comb_v[b][r, sl]

        fetch(0, 0)
        fetch(1, 1)

        def group_body(g, _):
            for b in range(2):
                c = 2 * g + b
                base = w_base + c * CHUNK
                wait_gathers(b, c)

                @pl.when(c >= 2)
                def _():
                    wait_wb(b, base - 2 * CHUNK)

                for h in range(2):
                    add_rows(b, h * H, (h + 1) * H)
                    pltpu.async_copy(
                        out_v[b].at[pl.ds(h * H, H)],
                        out_hbm.at[pl.ds(base + h * H, H)], sem_w[b])

                @pl.when(c + 2 < n_chunks)
                def _():
                    fetch(b, c + 2)
            return 0

        lax.fori_loop(0, n_chunks // 2, group_body, 0, unroll=False)

        wait_wb(0, w_base + (n_chunks - 2) * CHUNK)
        wait_wb(1, w_base + (n_chunks - 1) * CHUNK)

    return sc_kernel


def kernel(sequence, segment_label, token_table, segment_table):
    B, L = sequence.shape
    V, D = token_table.shape
    N = B * L
    rows_per_w = N // NW

    pe = _sinusoidal_pe(L, D)
    combo = (pe[:, None, :] + segment_table[None, :, :]).reshape(L * 3, D)
    seq_flat = sequence.reshape(-1).astype(jnp.int32)
    seg_flat = segment_label.reshape(-1).astype(jnp.int32)

    out = _make_sc_kernel(N, D, rows_per_w, L)(
        token_table, combo, seq_flat, seg_flat)
    return out.reshape(B, L, D)


# speedup vs baseline: 1.0152x; 1.0012x over previous
import functools
import math

import jax
import jax.numpy as jnp
from jax import lax
from jax.experimental import pallas as pl
from jax.experimental.pallas import tpu as pltpu
from jax.experimental.pallas import tpu_sc as plsc

NC, NS, LANES = 2, 16, 16
NW = NC * NS
CHUNK = 128


def _sinusoidal_pe(seq_len, d_model):
    position = jnp.arange(seq_len, dtype=jnp.float32)[:, None]
    div_term = jnp.exp(
        jnp.arange(0, d_model, 2, dtype=jnp.float32)
        * -(math.log(10000.0) / d_model))
    angles = position * div_term[None, :]
    pe = jnp.zeros((seq_len, d_model), dtype=jnp.float32)
    pe = pe.at[:, 0::2].set(jnp.sin(angles))
    pe = pe.at[:, 1::2].set(jnp.cos(angles))
    return pe


def _make_sc_kernel(N, D, rows_per_w, SEQ_LEN):
    n_chunks = rows_per_w // CHUNK
    assert n_chunks % 2 == 0
    mesh = plsc.VectorSubcoreMesh(core_axis_name="c", subcore_axis_name="s")

    @functools.partial(
        pl.kernel,
        mesh=mesh,
        out_type=jax.ShapeDtypeStruct((N, D), jnp.float32),
        scratch_types=[
            pltpu.VMEM((rows_per_w,), jnp.int32),
            pltpu.VMEM((rows_per_w,), jnp.int32),
            pltpu.VMEM((CHUNK, D), jnp.float32),
            pltpu.VMEM((CHUNK, D), jnp.float32),
            pltpu.VMEM((CHUNK, D), jnp.float32),
            pltpu.VMEM((CHUNK, D), jnp.float32),
            pltpu.VMEM((CHUNK, D), jnp.float32),
            pltpu.VMEM((CHUNK, D), jnp.float32),
            pltpu.SemaphoreType.DMA,
            pltpu.SemaphoreType.DMA,
            pltpu.SemaphoreType.DMA,
            pltpu.SemaphoreType.DMA,
            pltpu.SemaphoreType.DMA,
            pltpu.SemaphoreType.DMA,
            pltpu.VMEM_SHARED((600, 128), jnp.float32),
        ],
    )
    def sc_kernel(tok_hbm, combo_hbm, seq_hbm, seg_hbm, out_hbm,
                  seq_all, cidx_all, tok0, tok1, comb0, comb1,
                  out0, out1, st0, st1, sc0, sc1, sw0, sw1, combo_sh):
        tok_v = (tok0, tok1)
        comb_v = (comb0, comb1)
        out_v = (out0, out1)
        sem_t = (st0, st1)
        sem_c = (sc0, sc1)
        sem_w = (sw0, sw1)

        wid = lax.axis_index("s") * NC + lax.axis_index("c")
        w_base = wid * rows_per_w

        pltpu.async_copy(seq_hbm.at[pl.ds(w_base, rows_per_w)], seq_all,
                         sem_t[0])
        pltpu.async_copy(seg_hbm.at[pl.ds(w_base, rows_per_w)], cidx_all,
                         sem_t[1])

        @pl.when(lax.axis_index("s") == 0)
        def _():
            pltpu.sync_copy(combo_hbm, combo_sh)

        pltpu.make_async_copy(seq_hbm.at[pl.ds(w_base, rows_per_w)], seq_all,
                              sem_t[0]).wait()
        pltpu.make_async_copy(seg_hbm.at[pl.ds(w_base, rows_per_w)], cidx_all,
                              sem_t[1]).wait()
        plsc.subcore_barrier()

        iota16 = jax.lax.iota(jnp.int32, LANES)
        H = CHUNK // 2

        def fetch(b, ci):
            sl = pl.ds(ci * CHUNK, CHUNK)
            lo = pl.ds(ci * CHUNK, H)
            hi = pl.ds(ci * CHUNK + H, H)
            pltpu.async_copy(tok_hbm.at[seq_all.at[lo]],
                             tok_v[b].at[pl.ds(0, H)], sem_t[b])
            pltpu.async_copy(tok_hbm.at[seq_all.at[hi]],
                             tok_v[b].at[pl.ds(H, H)], sem_t[b])
            for q in range(CHUNK // LANES):
                qsl = pl.ds(ci * CHUNK + q * LANES, LANES)
                pos = ((w_base + ci * CHUNK + q * LANES) + iota16) % SEQ_LEN
                cidx_all[qsl] = pos * 3 + cidx_all[qsl]
            pltpu.async_copy(combo_sh.at[cidx_all.at[sl]], comb_v[b],
                             sem_c[b])

        def wait_gathers(b, ci):
            sl = pl.ds(ci * CHUNK, CHUNK)
            lo = pl.ds(ci * CHUNK, H)
            hi = pl.ds(ci * CHUNK + H, H)
            pltpu.make_async_copy(tok_hbm.at[seq_all.at[lo]],
                                  tok_v[b].at[pl.ds(0, H)], sem_t[b]).wait()
            pltpu.make_async_copy(tok_hbm.at[seq_all.at[hi]],
                                  tok_v[b].at[pl.ds(H, H)], sem_t[b]).wait()
            pltpu.make_async_copy(combo_sh.at[cidx_all.at[sl]], comb_v[b],
                                  sem_c[b]).wait()

        def wait_wb(b, base):
            pltpu.make_async_copy(out_v[b], out_hbm.at[pl.ds(base, CHUNK)],
                                  sem_w[b]).wait()

        def add_rows(b, lo, hi):
            @plsc.parallel_loop(lo, hi)
            def row_body(r):
                for j in range(D // LANES):
                    sl = pl.ds(j * LANES, LANES)
                    out_v[b][r, sl] = tok_v[b][r, sl] + comb_v[b][r, sl]

        fetch(0, 0)
        fetch(1, 1)

        def group_body(g, _):
            for b in range(2):
                c = 2 * g + b
                base = w_base + c * CHUNK
                wait_gathers(b, c)

                @pl.when(c >= 2)
                def _():
                    wait_wb(b, base - 2 * CHUNK)

                for h in range(2):
                    add_rows(b, h * H, (h + 1) * H)
                    pltpu.async_copy(
                        out_v[b].at[pl.ds(h * H, H)],
                        out_hbm.at[pl.ds(base + h * H, H)], sem_w[b])

                @pl.when(c + 2 < n_chunks)
                def _():
                    fetch(b, c + 2)
            return 0

        lax.fori_loop(0, n_chunks // 2, group_body, 0, unroll=False)

        wait_wb(0, w_base + (n_chunks - 2) * CHUNK)
        wait_wb(1, w_base + (n_chunks - 1) * CHUNK)

    return sc_kernel


def kernel(sequence, segment_label, token_table, segment_table):
    B, L = sequence.shape
    V, D = token_table.shape
    N = B * L
    rows_per_w = N // NW

    pe = _sinusoidal_pe(L, D)
    combo = (pe[:, None, :] + segment_table[None, :, :]).reshape(L * 3, D)
    seq_flat = sequence.reshape(-1).astype(jnp.int32)
    seg_flat = segment_label.reshape(-1).astype(jnp.int32)

    out = _make_sc_kernel(N, D, rows_per_w, L)(
        token_table, combo, seq_flat, seg_flat)
    return out.reshape(B, L, D)
